# initial kernel scaffold (unmeasured)
import jax
import jax.numpy as jnp
from jax import lax
from jax.experimental import pallas as pl
from jax.experimental.pallas import tpu as pltpu


def kernel(
    x,
):
    def body(*refs):
        pass

    out_shape = jax.ShapeDtypeStruct(..., jnp.float32)
    return pl.pallas_call(body, out_shape=out_shape)(...)


# baseline (device time: 14462 ns/iter reference)
import jax
import jax.numpy as jnp
from jax import lax
from jax.experimental import pallas as pl
from jax.experimental.pallas import tpu as pltpu

N_Z = 4


def kernel(x):
    m, n = x.shape

    def body(x_ref, out_ref, recv_buf, send_sems, recv_sems):
        my_x = lax.axis_index("x")
        my_y = lax.axis_index("y")
        my_z = lax.axis_index("z")

        barrier_sem = pltpu.get_barrier_semaphore()
        for d in (1, 2, 3):
            pl.semaphore_signal(
                barrier_sem,
                inc=1,
                device_id=(my_x, my_y, (my_z + d) % N_Z),
                device_id_type=pl.DeviceIdType.MESH,
            )
        pl.semaphore_wait(barrier_sem, 3)

        copies = []
        for d in (1, 2, 3):
            slot = 3 - d
            rdma = pltpu.make_async_remote_copy(
                src_ref=x_ref,
                dst_ref=recv_buf.at[slot],
                send_sem=send_sems.at[d - 1],
                recv_sem=recv_sems.at[slot],
                device_id=(my_x, my_y, (my_z + d) % N_Z),
                device_id_type=pl.DeviceIdType.MESH,
            )
            rdma.start()
            copies.append(rdma)
        for rdma in copies:
            rdma.wait()

        out_ref[:, :] = (
            x_ref[:, :]
            + recv_buf[0, :, :]
            + recv_buf[1, :, :]
            + recv_buf[2, :, :]
        )

    return pl.pallas_call(
        body,
        out_shape=jax.ShapeDtypeStruct((m, n), x.dtype),
        in_specs=[pl.BlockSpec(memory_space=pltpu.VMEM)],
        out_specs=pl.BlockSpec(memory_space=pltpu.VMEM),
        scratch_shapes=[
            pltpu.VMEM((N_Z - 1, m, n), x.dtype),
            pltpu.SemaphoreType.DMA((N_Z - 1,)),
            pltpu.SemaphoreType.DMA((N_Z - 1,)),
        ],
        compiler_params=pltpu.CompilerParams(collective_id=0),
    )(x)
